# baseline (speedup 1.0000x reference)
import functools

import jax
import jax.numpy as jnp
from jax import lax
from jax.experimental import pallas as pl
from jax.experimental.pallas import tpu as pltpu
from jax.experimental.pallas import tpu_sc as plsc

_NUM_CORES = 2
_NUM_SUBCORES = 16
_NUM_WORKERS = _NUM_CORES * _NUM_SUBCORES


@functools.lru_cache(maxsize=None)
def _build(seq_len: int, d_model: int):
    assert seq_len % _NUM_WORKERS == 0
    rows_per_worker = seq_len // _NUM_WORKERS
    chunk = min(16, rows_per_worker)
    n_chunks = rows_per_worker // chunk
    n_buf = min(6, n_chunks)

    mesh = plsc.VectorSubcoreMesh(
        core_axis_name="c", subcore_axis_name="s", num_cores=_NUM_CORES
    )

    @functools.partial(
        pl.kernel,
        mesh=mesh,
        out_type=jax.ShapeDtypeStruct((seq_len, d_model), jnp.float32),
        scratch_types=[
            [pltpu.VMEM((chunk, d_model), jnp.float32) for _ in range(n_buf)],
            [pltpu.SemaphoreType.DMA for _ in range(n_buf)],
            [pltpu.SemaphoreType.DMA for _ in range(n_buf)],
            pltpu.VMEM_SHARED((_NUM_SUBCORES, 64, d_model), jnp.float32),
            pltpu.SemaphoreType.DMA,
        ],
    )
    def copy_rows(table_hbm, out_hbm, bufs, gsems, ssems, shared, shsem):
        wid = lax.axis_index("s") * _NUM_CORES + lax.axis_index("c")
        sid = lax.axis_index("s")
        base = wid * rows_per_worker
        half = rows_per_worker // 2
        for p in range(2):
            pltpu.async_copy(
                table_hbm.at[pl.ds(base + p * half, half)], shared.at[sid], shsem
            ).wait()
            pltpu.async_copy(
                shared.at[sid], out_hbm.at[pl.ds(base + p * half, half)], shsem
            ).wait()
        return

        def src(i):
            return table_hbm.at[pl.ds(base + i * chunk, chunk)]

        def dst(i):
            return out_hbm.at[pl.ds(base + i * chunk, chunk)]

        gp = [None] * n_buf
        sp = [None] * n_buf
        for i in range(n_buf):
            gp[i] = pltpu.async_copy(src(i), bufs[i], gsems[i])
        for i in range(n_chunks):
            k = i % n_buf
            gp[k].wait()
            sp[k] = pltpu.async_copy(bufs[k], dst(i), ssems[k])
            j = i + n_buf
            if j < n_chunks:
                sp[k].wait()
                gp[k] = pltpu.async_copy(src(j), bufs[k], gsems[k])
                sp[k] = None
        for p in sp:
            if p is not None:
                p.wait()

    return copy_rows


def kernel(x, table):
    seq_len = x.shape[1]
    out = _build(seq_len, table.shape[1])(table)
    return out[None, :, :]


# speedup vs baseline: 1.0535x; 1.0535x over previous
import functools

import jax
import jax.numpy as jnp
from jax import lax
from jax.experimental import pallas as pl
from jax.experimental.pallas import tpu as pltpu
from jax.experimental.pallas import tpu_sc as plsc

_NUM_CORES = 2
_NUM_SUBCORES = 16
_NUM_WORKERS = _NUM_CORES * _NUM_SUBCORES

_CHUNK = 16
_STREAM_CHUNKS = 4
_DMA_CHUNKS = 4
_N_BUF = 4


@functools.lru_cache(maxsize=None)
def _build(seq_len: int, d_model: int):
    rows_per_worker = seq_len // _NUM_WORKERS
    assert seq_len % _NUM_WORKERS == 0
    assert rows_per_worker == (_STREAM_CHUNKS + _DMA_CHUNKS) * _CHUNK
    dma_half = _DMA_CHUNKS * _CHUNK // 2

    mesh = plsc.VectorSubcoreMesh(
        core_axis_name="c", subcore_axis_name="s", num_cores=_NUM_CORES
    )

    @functools.partial(
        pl.kernel,
        mesh=mesh,
        out_type=jax.ShapeDtypeStruct((seq_len, d_model), jnp.float32),
        scratch_types=[
            [pltpu.VMEM((_CHUNK, d_model), jnp.float32) for _ in range(_N_BUF)],
            [pltpu.SemaphoreType.DMA for _ in range(_N_BUF)],
            [pltpu.SemaphoreType.DMA for _ in range(_N_BUF)],
            pltpu.VMEM_SHARED(
                (_NUM_SUBCORES, 2, dma_half, d_model), jnp.float32
            ),
            [pltpu.SemaphoreType.DMA for _ in range(2)],
        ],
    )
    def copy_rows(table_hbm, out_hbm, bufs, gsems, ssems, shared, dsems):
        wid = lax.axis_index("s") * _NUM_CORES + lax.axis_index("c")
        sid = lax.axis_index("s")
        base = wid * rows_per_worker
        sbase = base + 2 * dma_half

        def src(i):
            return table_hbm.at[pl.ds(sbase + i * _CHUNK, _CHUNK)]

        def dst(i):
            return out_hbm.at[pl.ds(sbase + i * _CHUNK, _CHUNK)]

        def dsrc(p):
            return table_hbm.at[pl.ds(base + p * dma_half, dma_half)]

        def ddst(p):
            return out_hbm.at[pl.ds(base + p * dma_half, dma_half)]

        dg0 = pltpu.async_copy(dsrc(0), shared.at[sid, 0], dsems[0])
        gp = [None] * _N_BUF
        sp = [None] * _N_BUF
        for i in range(min(_N_BUF, _STREAM_CHUNKS)):
            gp[i] = pltpu.async_copy(src(i), bufs[i], gsems[i])
        dg1 = pltpu.async_copy(dsrc(1), shared.at[sid, 1], dsems[1])

        for i in range(_STREAM_CHUNKS):
            k = i % _N_BUF
            gp[k].wait()
            sp[k] = pltpu.async_copy(bufs[k], dst(i), ssems[k])
            j = i + _N_BUF
            if j < _STREAM_CHUNKS:
                sp[k].wait()
                gp[k] = pltpu.async_copy(src(j), bufs[k], gsems[k])
                sp[k] = None
            if i == _STREAM_CHUNKS // 2 - 1:
                dg0.wait()
                pltpu.async_copy(shared.at[sid, 0], ddst(0), dsems[0])

        dg1.wait()
        ds1 = pltpu.async_copy(shared.at[sid, 1], ddst(1), dsems[1])
        pltpu.make_async_copy(shared.at[sid, 0], ddst(0), dsems[0]).wait()
        ds1.wait()
        for p in sp:
            if p is not None:
                p.wait()

    return copy_rows


def kernel(x, table):
    seq_len = x.shape[1]
    out = _build(seq_len, table.shape[1])(table)
    return out[None, :, :]
